# baseline (speedup 1.0000x reference)
import functools

import jax
import jax.numpy as jnp
from jax import lax
from jax.experimental import pallas as pl
from jax.experimental.pallas import tpu as pltpu
from jax.experimental.pallas import tpu_sc as plsc

_H = 128
_NUM_CAND = 1000
_PC = 2048
_POS_OFF = 1024
_BI = 256
_CJ = 512
_BE = 2000
_NC, _NS = 2, 16
_NW = _NC * _NS
_CH = 80


def _precompute_body(ne, fcol, w0a, w0b, ws, wd, b0r, a_out, b_out):
    x = ne[...]
    f = fcol[...]
    a_out[...] = (jnp.dot(x, w0a[...], preferred_element_type=jnp.float32)
                  + f * ws[...] + b0r[...]).astype(jnp.bfloat16)
    b_out[...] = (jnp.dot(x, w0b[...], preferred_element_type=jnp.float32)
                  + f * wd[...]).astype(jnp.bfloat16)


def _mlp_body(ha, hb, w1, b1r, w2, b2r, out):
    h0 = jnp.maximum(ha[...] + hb[...], jnp.bfloat16(0.0))
    h1 = jnp.maximum(
        jnp.dot(h0, w1[...], preferred_element_type=jnp.float32) + b1r[...], 0.0)
    out[...] = jnp.dot(h1.astype(jnp.bfloat16), w2[...],
                       preferred_element_type=jnp.float32) + b2r[...]


def _rank_body(fi_ref, f2d_ref, pos_ref, sel_ref, *, n_j):
    i = pl.program_id(0)
    fi = fi_ref[...]
    ii = i * _BI + lax.broadcasted_iota(jnp.int32, (_BI, 1), 0)

    def jstep(j, rank):
        fj = f2d_ref[pl.ds(j, 1), :]
        jidx = j * _CJ + lax.broadcasted_iota(jnp.int32, (1, _CJ), 1)
        lt = fj < fi
        tie = (fj == fi) & (jidx < ii)
        return rank + jnp.sum((lt | tie).astype(jnp.int32), axis=1, keepdims=True)

    rank = lax.fori_loop(0, n_j, jstep, jnp.zeros((_BI, 1), jnp.int32))

    @pl.when(i == 0)
    def _():
        sel_ref[...] = jnp.zeros((1, _PC), jnp.int32)

    for c in range(_PC // 128):
        sl = slice(c * 128, (c + 1) * 128)
        match = rank == pos_ref[:, sl]
        vals = jnp.where(match, ii, 0)
        sel_ref[:, sl] = sel_ref[:, sl] + jnp.sum(vals, axis=0, keepdims=True)


def _make_sc_gather(n_edges, n_lanes, dtype):
    per_w = n_edges // _NW
    n_ch = per_w // _CH
    mesh = plsc.VectorSubcoreMesh(core_axis_name="c", subcore_axis_name="s")

    @functools.partial(
        pl.kernel,
        mesh=mesh,
        out_type=(jax.ShapeDtypeStruct((n_edges, n_lanes), dtype),
                  jax.ShapeDtypeStruct((n_edges, n_lanes), dtype)),
        scratch_types=[
            pltpu.VMEM((_CH,), jnp.int32),
            pltpu.VMEM((_CH,), jnp.int32),
            pltpu.VMEM((_CH, n_lanes), dtype),
            pltpu.VMEM((_CH, n_lanes), dtype),
            pltpu.SemaphoreType.DMA,
            pltpu.SemaphoreType.DMA,
        ],
        compiler_params=pltpu.CompilerParams(use_tc_tiling_on_sc=False),
    )
    def gather_k(a_hbm, b_hbm, src_hbm, dst_hbm, oa_hbm, ob_hbm,
                 si, di, ra, rb, sa, sb):
        wid = lax.axis_index("s") * _NC + lax.axis_index("c")
        base = wid * per_w

        def step(ci, carry):
            off = base + ci * _CH
            pltpu.sync_copy(src_hbm.at[pl.ds(off, _CH)], si)
            pltpu.sync_copy(dst_hbm.at[pl.ds(off, _CH)], di)
            ca = pltpu.async_copy(a_hbm.at[si], ra, sa)
            cb = pltpu.async_copy(b_hbm.at[di], rb, sb)
            ca.wait()
            cb.wait()
            pltpu.sync_copy(ra, oa_hbm.at[pl.ds(off, _CH)])
            pltpu.sync_copy(rb, ob_hbm.at[pl.ds(off, _CH)])
            return carry

        lax.fori_loop(0, n_ch, step, 0)

    return gather_k


def kernel(node_embeddings, edge_index, num_nodes, fiedler_vector,
           W0, b0, W1, b1, W2, b2):
    n, h = node_embeddings.shape
    n_edges = edge_index.shape[1]
    f32 = jnp.float32

    fcol = fiedler_vector.reshape(n, 1)
    w0a = W0[:h]
    w0b = W0[h:2 * h]
    ws = W0[2 * h].reshape(1, h)
    wd = W0[2 * h + 1].reshape(1, h)
    bf16 = jnp.bfloat16
    a_tab, b_tab = pl.pallas_call(
        _precompute_body,
        out_shape=(jax.ShapeDtypeStruct((n, h), bf16),
                   jax.ShapeDtypeStruct((n, h), bf16)),
    )(node_embeddings, fcol, w0a, w0b, ws, wd, b0.reshape(1, h))

    nw = h // 2
    a_pack = lax.bitcast_convert_type(a_tab.reshape(n, nw, 2), jnp.int32)
    b_pack = lax.bitcast_convert_type(b_tab.reshape(n, nw, 2), jnp.int32)
    src = edge_index[0]
    dst = edge_index[1]
    ga_p, gb_p = _make_sc_gather(n_edges, nw, jnp.int32)(a_pack, b_pack, src, dst)
    ga = lax.bitcast_convert_type(ga_p, bf16).reshape(n_edges, h)
    gb = lax.bitcast_convert_type(gb_p, bf16).reshape(n_edges, h)

    n_blk = n_edges // _BE
    scores2d = pl.pallas_call(
        _mlp_body,
        grid=(n_blk,),
        in_specs=[
            pl.BlockSpec((_BE, h), lambda i: (i, 0)),
            pl.BlockSpec((_BE, h), lambda i: (i, 0)),
            pl.BlockSpec((h, h), lambda i: (0, 0)),
            pl.BlockSpec((1, h), lambda i: (0, 0)),
            pl.BlockSpec((h, 1), lambda i: (0, 0)),
            pl.BlockSpec((1, 1), lambda i: (0, 0)),
        ],
        out_specs=pl.BlockSpec((_BE, 1), lambda i: (i, 0)),
        out_shape=jax.ShapeDtypeStruct((n_edges, 1), f32),
    )(ga, gb, W1.astype(bf16), b1.reshape(1, h), W2.astype(bf16),
      b2.reshape(1, 1))
    edge_scores = scores2d.reshape(n_edges)

    np_pad = ((n + _CJ - 1) // _CJ) * _CJ
    fpad = jnp.concatenate(
        [fiedler_vector, jnp.full((np_pad - n,), jnp.inf, f32)])
    f2d = fpad.reshape(np_pad // _CJ, _CJ)
    num_pairs = min(_NUM_CAND, n * (n - 1) // 4)
    ck = jax.random.key(42)
    k1, k2 = jax.random.split(ck)
    idx1 = jax.random.randint(k1, (num_pairs,), 0, num_nodes // 2, jnp.int32)
    idx2 = jax.random.randint(k2, (num_pairs,), num_nodes // 2, num_nodes,
                              jnp.int32)
    pos = jnp.full((1, _PC), -1, jnp.int32)
    pos = pos.at[0, :num_pairs].set(idx1)
    pos = pos.at[0, _POS_OFF:_POS_OFF + num_pairs].set(idx2)

    sel = pl.pallas_call(
        functools.partial(_rank_body, n_j=np_pad // _CJ),
        grid=(np_pad // _BI,),
        in_specs=[
            pl.BlockSpec((_BI, 1), lambda i: (i, 0)),
            pl.BlockSpec((np_pad // _CJ, _CJ), lambda i: (0, 0)),
            pl.BlockSpec((1, _PC), lambda i: (0, 0)),
        ],
        out_specs=pl.BlockSpec((1, _PC), lambda i: (0, 0)),
        out_shape=jax.ShapeDtypeStruct((1, _PC), jnp.int32),
    )(fpad.reshape(np_pad, 1), f2d, pos)

    src_c = sel[0, :num_pairs]
    dst_c = sel[0, _POS_OFF:_POS_OFF + num_pairs]
    candidate_edges = jnp.stack([src_c, dst_c], axis=0)
    return edge_scores, candidate_edges


# speedup vs baseline: 4.1536x; 4.1536x over previous
import functools

import jax
import jax.numpy as jnp
from jax import lax
from jax.experimental import pallas as pl
from jax.experimental.pallas import tpu as pltpu
from jax.experimental.pallas import tpu_sc as plsc

_H = 128
_NUM_CAND = 1000
_PC = 2048
_POS_OFF = 1024
_BI = 256
_CJ = 512
_BE = 2000
_NC, _NS = 2, 16
_NW = _NC * _NS
_CH = 80


def _precompute_body(ne, fcol, w0a, w0b, ws, wd, b0r, a_out, b_out):
    x = ne[...]
    f = fcol[...]
    a_out[...] = (jnp.dot(x, w0a[...], preferred_element_type=jnp.float32)
                  + f * ws[...] + b0r[...])
    b_out[...] = (jnp.dot(x, w0b[...], preferred_element_type=jnp.float32)
                  + f * wd[...])


def _mlp_body(h0, w1, b1r, w2, b2r, out):
    hb = h0[...].astype(jnp.bfloat16)
    h1 = jnp.maximum(
        jnp.dot(hb, w1[...], preferred_element_type=jnp.float32) + b1r[...], 0.0)
    out[...] = jnp.dot(h1.astype(jnp.bfloat16), w2[...],
                       preferred_element_type=jnp.float32) + b2r[...]


def _rank_body(fi_ref, f2d_ref, pos_ref, sel_ref, *, n_j):
    i = pl.program_id(0)
    fi = fi_ref[...]
    ii = i * _BI + lax.broadcasted_iota(jnp.int32, (_BI, 1), 0)

    def jstep(j, rank):
        fj = f2d_ref[pl.ds(j, 1), :]
        jidx = j * _CJ + lax.broadcasted_iota(jnp.int32, (1, _CJ), 1)
        lt = fj < fi
        tie = (fj == fi) & (jidx < ii)
        return rank + jnp.sum((lt | tie).astype(jnp.int32), axis=1, keepdims=True)

    rank = lax.fori_loop(0, n_j, jstep, jnp.zeros((_BI, 1), jnp.int32))

    @pl.when(i == 0)
    def _():
        sel_ref[...] = jnp.zeros((1, _PC), jnp.int32)

    for c in range(_PC // 128):
        sl = slice(c * 128, (c + 1) * 128)
        match = rank == pos_ref[:, sl]
        vals = jnp.where(match, ii, 0)
        sel_ref[:, sl] = sel_ref[:, sl] + jnp.sum(vals, axis=0, keepdims=True)


def _make_sc_gather(n_edges):
    per_w = n_edges // _NW
    n_ch = per_w // _CH
    assert n_ch % 2 == 1 and n_ch >= 3
    mesh = plsc.VectorSubcoreMesh(core_axis_name="c", subcore_axis_name="s")

    @functools.partial(
        pl.kernel,
        mesh=mesh,
        out_type=jax.ShapeDtypeStruct((n_edges, _H), jnp.float32),
        scratch_types=[
            pltpu.VMEM((_CH,), jnp.int32), pltpu.VMEM((_CH,), jnp.int32),
            pltpu.VMEM((_CH,), jnp.int32), pltpu.VMEM((_CH,), jnp.int32),
            pltpu.VMEM((_CH, _H), jnp.float32), pltpu.VMEM((_CH, _H), jnp.float32),
            pltpu.VMEM((_CH, _H), jnp.float32), pltpu.VMEM((_CH, _H), jnp.float32),
            pltpu.VMEM((_CH, _H), jnp.float32), pltpu.VMEM((_CH, _H), jnp.float32),
            pltpu.SemaphoreType.DMA, pltpu.SemaphoreType.DMA,
            pltpu.SemaphoreType.DMA, pltpu.SemaphoreType.DMA,
            pltpu.SemaphoreType.DMA, pltpu.SemaphoreType.DMA,
        ],
    )
    def gather_k(a_hbm, b_hbm, src_hbm, dst_hbm, out_hbm,
                 si0, di0, si1, di1, ra0, rb0, ra1, rb1, hv0, hv1,
                 sa0, sb0, sa1, sb1, so0, so1):
        wid = lax.axis_index("s") * _NC + lax.axis_index("c")
        base = wid * per_w

        bufs = ((si0, di0, ra0, rb0, hv0, sa0, sb0, so0),
                (si1, di1, ra1, rb1, hv1, sa1, sb1, so1))

        def issue(c, p):
            si, di, ra, rb, _, sa, sb, _ = bufs[p]
            off = base + c * _CH
            pltpu.sync_copy(src_hbm.at[pl.ds(off, _CH)], si)
            pltpu.sync_copy(dst_hbm.at[pl.ds(off, _CH)], di)
            pltpu.async_copy(a_hbm.at[si], ra, sa)
            pltpu.async_copy(b_hbm.at[di], rb, sb)

        def process(c, p):
            si, di, ra, rb, hv, sa, sb, so = bufs[p]
            pltpu.make_async_copy(a_hbm.at[si], ra, sa).wait()
            pltpu.make_async_copy(b_hbm.at[di], rb, sb).wait()

            def ebody(e, carry):
                for u in range(_H // 16):
                    sl = pl.ds(u * 16, 16)
                    hv[e, sl] = jnp.maximum(ra[e, sl] + rb[e, sl], 0.0)
                return carry

            lax.fori_loop(0, _CH, ebody, 0)
            off = base + c * _CH
            pltpu.sync_copy(hv, out_hbm.at[pl.ds(off, _CH)])

        issue(0, 0)

        def pair(k, carry):
            c0 = 2 * k
            issue(c0 + 1, 1)
            process(c0, 0)
            issue(c0 + 2, 0)
            process(c0 + 1, 1)
            return carry

        lax.fori_loop(0, (n_ch - 1) // 2, pair, 0)
        process(n_ch - 1, 0)

    return gather_k


def kernel(node_embeddings, edge_index, num_nodes, fiedler_vector,
           W0, b0, W1, b1, W2, b2):
    n, h = node_embeddings.shape
    n_edges = edge_index.shape[1]
    f32 = jnp.float32

    fcol = fiedler_vector.reshape(n, 1)
    w0a = W0[:h]
    w0b = W0[h:2 * h]
    ws = W0[2 * h].reshape(1, h)
    wd = W0[2 * h + 1].reshape(1, h)
    bf16 = jnp.bfloat16
    a_tab, b_tab = pl.pallas_call(
        _precompute_body,
        out_shape=(jax.ShapeDtypeStruct((n, h), f32),
                   jax.ShapeDtypeStruct((n, h), f32)),
    )(node_embeddings, fcol, w0a, w0b, ws, wd, b0.reshape(1, h))

    src = edge_index[0]
    dst = edge_index[1]
    h0 = _make_sc_gather(n_edges)(a_tab, b_tab, src, dst)

    n_blk = n_edges // _BE
    scores2d = pl.pallas_call(
        _mlp_body,
        grid=(n_blk,),
        in_specs=[
            pl.BlockSpec((_BE, h), lambda i: (i, 0)),
            pl.BlockSpec((h, h), lambda i: (0, 0)),
            pl.BlockSpec((1, h), lambda i: (0, 0)),
            pl.BlockSpec((h, 1), lambda i: (0, 0)),
            pl.BlockSpec((1, 1), lambda i: (0, 0)),
        ],
        out_specs=pl.BlockSpec((_BE, 1), lambda i: (i, 0)),
        out_shape=jax.ShapeDtypeStruct((n_edges, 1), f32),
    )(h0, W1.astype(bf16), b1.reshape(1, h), W2.astype(bf16),
      b2.reshape(1, 1))
    edge_scores = scores2d.reshape(n_edges)

    np_pad = ((n + _CJ - 1) // _CJ) * _CJ
    fpad = jnp.concatenate(
        [fiedler_vector, jnp.full((np_pad - n,), jnp.inf, f32)])
    f2d = fpad.reshape(np_pad // _CJ, _CJ)
    num_pairs = min(_NUM_CAND, n * (n - 1) // 4)
    ck = jax.random.key(42)
    k1, k2 = jax.random.split(ck)
    idx1 = jax.random.randint(k1, (num_pairs,), 0, num_nodes // 2, jnp.int32)
    idx2 = jax.random.randint(k2, (num_pairs,), num_nodes // 2, num_nodes,
                              jnp.int32)
    pos = jnp.full((1, _PC), -1, jnp.int32)
    pos = pos.at[0, :num_pairs].set(idx1)
    pos = pos.at[0, _POS_OFF:_POS_OFF + num_pairs].set(idx2)

    sel = pl.pallas_call(
        functools.partial(_rank_body, n_j=np_pad // _CJ),
        grid=(np_pad // _BI,),
        in_specs=[
            pl.BlockSpec((_BI, 1), lambda i: (i, 0)),
            pl.BlockSpec((np_pad // _CJ, _CJ), lambda i: (0, 0)),
            pl.BlockSpec((1, _PC), lambda i: (0, 0)),
        ],
        out_specs=pl.BlockSpec((1, _PC), lambda i: (0, 0)),
        out_shape=jax.ShapeDtypeStruct((1, _PC), jnp.int32),
    )(fpad.reshape(np_pad, 1), f2d, pos)

    src_c = sel[0, :num_pairs]
    dst_c = sel[0, _POS_OFF:_POS_OFF + num_pairs]
    candidate_edges = jnp.stack([src_c, dst_c], axis=0)
    return edge_scores, candidate_edges


# speedup vs baseline: 4.3664x; 1.0512x over previous
import functools

import jax
import jax.numpy as jnp
from jax import lax
from jax.experimental import pallas as pl
from jax.experimental.pallas import tpu as pltpu
from jax.experimental.pallas import tpu_sc as plsc

_H = 128
_NUM_CAND = 1000
_PC = 2048
_POS_OFF = 1024
_BI = 256
_CJ = 128
_BE = 2000
_NC, _NS = 2, 16
_NW = _NC * _NS
_CH = 80


def _precompute_body(ne, fcol, w0a, w0b, ws, wd, b0r, a_out, b_out):
    x = ne[...]
    f = fcol[...]
    a_out[...] = (jnp.dot(x, w0a[...], preferred_element_type=jnp.float32)
                  + f * ws[...] + b0r[...])
    b_out[...] = (jnp.dot(x, w0b[...], preferred_element_type=jnp.float32)
                  + f * wd[...])


def _mlp_body(h0, w1, b1r, w2, b2r, out):
    hb = h0[...].astype(jnp.bfloat16)
    h1 = jnp.maximum(
        jnp.dot(hb, w1[...], preferred_element_type=jnp.float32) + b1r[...], 0.0)
    out[...] = jnp.dot(h1.astype(jnp.bfloat16), w2[...],
                       preferred_element_type=jnp.float32) + b2r[...]


def _rank_body(fi_ref, f2d_ref, pos_ref, sel_ref, *, n_j):
    i = pl.program_id(0)
    fi = fi_ref[...]
    ii = i * _BI + lax.broadcasted_iota(jnp.int32, (_BI, 1), 0)
    d0 = i * (_BI // _CJ)

    def le_step(j, acc):
        return acc + (f2d_ref[pl.ds(j, 1), :] <= fi).astype(jnp.int32)

    def lt_step(j, acc):
        return acc + (f2d_ref[pl.ds(j, 1), :] < fi).astype(jnp.int32)

    acc = lax.fori_loop(0, d0, le_step, jnp.zeros((_BI, _CJ), jnp.int32))
    for t in range(_BI // _CJ):
        j = d0 + t
        fj = f2d_ref[pl.ds(j, 1), :]
        jidx = j * _CJ + lax.broadcasted_iota(jnp.int32, (1, _CJ), 1)
        tie = (fj == fi) & (jidx < ii)
        acc = acc + ((fj < fi) | tie).astype(jnp.int32)
    acc = lax.fori_loop(d0 + _BI // _CJ, n_j, lt_step, acc)
    rank = jnp.sum(acc, axis=1, keepdims=True)

    @pl.when(i == 0)
    def _():
        sel_ref[...] = jnp.zeros((1, _PC), jnp.int32)

    for c in range(_PC // 128):
        sl = slice(c * 128, (c + 1) * 128)
        match = rank == pos_ref[:, sl]
        vals = jnp.where(match, ii, 0)
        sel_ref[:, sl] = sel_ref[:, sl] + jnp.sum(vals, axis=0, keepdims=True)


def _make_sc_gather(n_edges):
    per_w = n_edges // _NW
    n_ch = per_w // _CH
    assert n_ch % 2 == 1 and n_ch >= 3
    mesh = plsc.VectorSubcoreMesh(core_axis_name="c", subcore_axis_name="s")

    @functools.partial(
        pl.kernel,
        mesh=mesh,
        out_type=jax.ShapeDtypeStruct((n_edges, _H), jnp.float32),
        scratch_types=[
            pltpu.VMEM((_CH,), jnp.int32), pltpu.VMEM((_CH,), jnp.int32),
            pltpu.VMEM((_CH,), jnp.int32), pltpu.VMEM((_CH,), jnp.int32),
            pltpu.VMEM((_CH, _H), jnp.float32), pltpu.VMEM((_CH, _H), jnp.float32),
            pltpu.VMEM((_CH, _H), jnp.float32), pltpu.VMEM((_CH, _H), jnp.float32),
            pltpu.VMEM((_CH, _H), jnp.float32), pltpu.VMEM((_CH, _H), jnp.float32),
            pltpu.SemaphoreType.DMA, pltpu.SemaphoreType.DMA,
            pltpu.SemaphoreType.DMA, pltpu.SemaphoreType.DMA,
            pltpu.SemaphoreType.DMA, pltpu.SemaphoreType.DMA,
        ],
    )
    def gather_k(a_hbm, b_hbm, src_hbm, dst_hbm, out_hbm,
                 si0, di0, si1, di1, ra0, rb0, ra1, rb1, hv0, hv1,
                 sa0, sb0, sa1, sb1, so0, so1):
        wid = lax.axis_index("s") * _NC + lax.axis_index("c")
        base = wid * per_w

        bufs = ((si0, di0, ra0, rb0, hv0, sa0, sb0, so0),
                (si1, di1, ra1, rb1, hv1, sa1, sb1, so1))

        def issue(c, p):
            si, di, ra, rb, _, sa, sb, _ = bufs[p]
            off = base + c * _CH
            pltpu.sync_copy(src_hbm.at[pl.ds(off, _CH)], si)
            pltpu.sync_copy(dst_hbm.at[pl.ds(off, _CH)], di)
            pltpu.async_copy(a_hbm.at[si], ra, sa)
            pltpu.async_copy(b_hbm.at[di], rb, sb)

        def process(c, p):
            si, di, ra, rb, hv, sa, sb, so = bufs[p]
            pltpu.make_async_copy(a_hbm.at[si], ra, sa).wait()
            pltpu.make_async_copy(b_hbm.at[di], rb, sb).wait()

            def ebody(e, carry):
                for u in range(_H // 16):
                    sl = pl.ds(u * 16, 16)
                    hv[e, sl] = jnp.maximum(ra[e, sl] + rb[e, sl], 0.0)
                return carry

            lax.fori_loop(0, _CH, ebody, 0)
            off = base + c * _CH
            pltpu.sync_copy(hv, out_hbm.at[pl.ds(off, _CH)])

        issue(0, 0)

        def pair(k, carry):
            c0 = 2 * k
            issue(c0 + 1, 1)
            process(c0, 0)
            issue(c0 + 2, 0)
            process(c0 + 1, 1)
            return carry

        lax.fori_loop(0, (n_ch - 1) // 2, pair, 0)
        process(n_ch - 1, 0)

    return gather_k


def kernel(node_embeddings, edge_index, num_nodes, fiedler_vector,
           W0, b0, W1, b1, W2, b2):
    n, h = node_embeddings.shape
    n_edges = edge_index.shape[1]
    f32 = jnp.float32

    fcol = fiedler_vector.reshape(n, 1)
    w0a = W0[:h]
    w0b = W0[h:2 * h]
    ws = W0[2 * h].reshape(1, h)
    wd = W0[2 * h + 1].reshape(1, h)
    bf16 = jnp.bfloat16
    a_tab, b_tab = pl.pallas_call(
        _precompute_body,
        out_shape=(jax.ShapeDtypeStruct((n, h), f32),
                   jax.ShapeDtypeStruct((n, h), f32)),
    )(node_embeddings, fcol, w0a, w0b, ws, wd, b0.reshape(1, h))

    src = edge_index[0]
    dst = edge_index[1]
    h0 = _make_sc_gather(n_edges)(a_tab, b_tab, src, dst)

    n_blk = n_edges // _BE
    scores2d = pl.pallas_call(
        _mlp_body,
        grid=(n_blk,),
        in_specs=[
            pl.BlockSpec((_BE, h), lambda i: (i, 0)),
            pl.BlockSpec((h, h), lambda i: (0, 0)),
            pl.BlockSpec((1, h), lambda i: (0, 0)),
            pl.BlockSpec((h, 1), lambda i: (0, 0)),
            pl.BlockSpec((1, 1), lambda i: (0, 0)),
        ],
        out_specs=pl.BlockSpec((_BE, 1), lambda i: (i, 0)),
        out_shape=jax.ShapeDtypeStruct((n_edges, 1), f32),
    )(h0, W1.astype(bf16), b1.reshape(1, h), W2.astype(bf16),
      b2.reshape(1, 1))
    edge_scores = scores2d.reshape(n_edges)

    np_pad = ((n + _BI - 1) // _BI) * _BI
    fpad = jnp.concatenate(
        [fiedler_vector, jnp.full((np_pad - n,), jnp.inf, f32)])
    f2d = fpad.reshape(np_pad // _CJ, _CJ)
    num_pairs = min(_NUM_CAND, n * (n - 1) // 4)
    ck = jax.random.key(42)
    k1, k2 = jax.random.split(ck)
    idx1 = jax.random.randint(k1, (num_pairs,), 0, num_nodes // 2, jnp.int32)
    idx2 = jax.random.randint(k2, (num_pairs,), num_nodes // 2, num_nodes,
                              jnp.int32)
    pos = jnp.full((1, _PC), -1, jnp.int32)
    pos = pos.at[0, :num_pairs].set(idx1)
    pos = pos.at[0, _POS_OFF:_POS_OFF + num_pairs].set(idx2)

    sel = pl.pallas_call(
        functools.partial(_rank_body, n_j=np_pad // _CJ),
        grid=(np_pad // _BI,),
        in_specs=[
            pl.BlockSpec((_BI, 1), lambda i: (i, 0)),
            pl.BlockSpec((np_pad // _CJ, _CJ), lambda i: (0, 0)),
            pl.BlockSpec((1, _PC), lambda i: (0, 0)),
        ],
        out_specs=pl.BlockSpec((1, _PC), lambda i: (0, 0)),
        out_shape=jax.ShapeDtypeStruct((1, _PC), jnp.int32),
    )(fpad.reshape(np_pad, 1), f2d, pos)

    src_c = sel[0, :num_pairs]
    dst_c = sel[0, _POS_OFF:_POS_OFF + num_pairs]
    candidate_edges = jnp.stack([src_c, dst_c], axis=0)
    return edge_scores, candidate_edges


# speedup vs baseline: 4.4151x; 1.0112x over previous
import functools

import numpy as np

import jax
import jax.numpy as jnp
from jax import lax
from jax.experimental import pallas as pl
from jax.experimental.pallas import tpu as pltpu
from jax.experimental.pallas import tpu_sc as plsc

_H = 128
_NUM_CAND = 1000
_PC = 2048
_POS_OFF = 1024
_BI = 256
_CJ = 128
_BE = 2000
_NC, _NS = 2, 16
_NW = _NC * _NS
_CH = 80


def _precompute_body(ne, fcol, w0a, w0b, ws, wd, b0r, a_out, b_out):
    x = ne[...]
    f = fcol[...]
    a_out[...] = (jnp.dot(x, w0a[...], preferred_element_type=jnp.float32)
                  + f * ws[...] + b0r[...])
    b_out[...] = (jnp.dot(x, w0b[...], preferred_element_type=jnp.float32)
                  + f * wd[...])


def _mlp_body(h0, w1, b1r, w2, b2r, out):
    hb = h0[...].astype(jnp.bfloat16)
    h1 = jnp.maximum(
        jnp.dot(hb, w1[...], preferred_element_type=jnp.float32) + b1r[...], 0.0)
    out[...] = jnp.dot(h1.astype(jnp.bfloat16), w2[...],
                       preferred_element_type=jnp.float32) + b2r[...]


def _rank_body(fi_ref, f2d_ref, pos_ref, sel_ref, *, n_j):
    i = pl.program_id(0)
    fi = fi_ref[...]
    ii = i * _BI + lax.broadcasted_iota(jnp.int32, (_BI, 1), 0)
    d0 = i * (_BI // _CJ)

    def le_step(j, acc):
        return acc + (f2d_ref[pl.ds(j, 1), :] <= fi).astype(jnp.int32)

    def lt_step(j, acc):
        return acc + (f2d_ref[pl.ds(j, 1), :] < fi).astype(jnp.int32)

    acc = lax.fori_loop(0, d0, le_step, jnp.zeros((_BI, _CJ), jnp.int32))
    for t in range(_BI // _CJ):
        j = d0 + t
        fj = f2d_ref[pl.ds(j, 1), :]
        jidx = j * _CJ + lax.broadcasted_iota(jnp.int32, (1, _CJ), 1)
        tie = (fj == fi) & (jidx < ii)
        acc = acc + ((fj < fi) | tie).astype(jnp.int32)
    acc = lax.fori_loop(d0 + _BI // _CJ, n_j, lt_step, acc)
    rank = jnp.sum(acc, axis=1, keepdims=True)

    @pl.when(i == 0)
    def _():
        sel_ref[...] = jnp.zeros((1, _PC), jnp.int32)

    for c in range(_PC // 128):
        sl = slice(c * 128, (c + 1) * 128)
        match = rank == pos_ref[:, sl]
        vals = jnp.where(match, ii, 0)
        sel_ref[:, sl] = sel_ref[:, sl] + jnp.sum(vals, axis=0, keepdims=True)


def _make_sc_gather(n_edges):
    per_w = n_edges // _NW
    n_ch = per_w // _CH
    assert n_ch % 2 == 1 and n_ch >= 5
    mesh = plsc.VectorSubcoreMesh(core_axis_name="c", subcore_axis_name="s")

    @functools.partial(
        pl.kernel,
        mesh=mesh,
        out_type=jax.ShapeDtypeStruct((n_edges, _H), jnp.float32),
        scratch_types=[
            pltpu.VMEM((per_w,), jnp.int32), pltpu.VMEM((per_w,), jnp.int32),
            pltpu.VMEM((_CH, _H), jnp.float32), pltpu.VMEM((_CH, _H), jnp.float32),
            pltpu.VMEM((_CH, _H), jnp.float32), pltpu.VMEM((_CH, _H), jnp.float32),
            pltpu.VMEM((_CH, _H), jnp.float32), pltpu.VMEM((_CH, _H), jnp.float32),
            pltpu.SemaphoreType.DMA, pltpu.SemaphoreType.DMA,
            pltpu.SemaphoreType.DMA, pltpu.SemaphoreType.DMA,
            pltpu.SemaphoreType.DMA, pltpu.SemaphoreType.DMA,
        ],
    )
    def gather_k(a_hbm, b_hbm, src_hbm, dst_hbm, out_hbm,
                 si_all, di_all, ra0, rb0, ra1, rb1, hv0, hv1,
                 sa0, sb0, sa1, sb1, so0, so1):
        wid = lax.axis_index("s") * _NC + lax.axis_index("c")
        base = wid * per_w
        pltpu.sync_copy(src_hbm.at[pl.ds(base, per_w)], si_all)
        pltpu.sync_copy(dst_hbm.at[pl.ds(base, per_w)], di_all)

        bufs = ((ra0, rb0, hv0, sa0, sb0, so0),
                (ra1, rb1, hv1, sa1, sb1, so1))

        def issue(c, p):
            ra, rb, _, sa, sb, _ = bufs[p]
            isl = pl.ds(c * _CH, _CH)
            pltpu.async_copy(a_hbm.at[si_all.at[isl]], ra, sa)
            pltpu.async_copy(b_hbm.at[di_all.at[isl]], rb, sb)

        def wait_gather(p):
            ra, rb, _, sa, sb, _ = bufs[p]
            pltpu.make_async_copy(a_hbm.at[si_all.at[pl.ds(0, _CH)]], ra, sa).wait()
            pltpu.make_async_copy(b_hbm.at[di_all.at[pl.ds(0, _CH)]], rb, sb).wait()

        def wait_store(p):
            _, _, hv, _, _, so = bufs[p]
            pltpu.make_async_copy(
                hv, out_hbm.at[pl.ds(base, _CH)], so).wait()

        def compute_store(c, p):
            ra, rb, hv, _, _, so = bufs[p]

            def ebody(e, carry):
                for u in range(_H // 16):
                    sl = pl.ds(u * 16, 16)
                    hv[e, sl] = jnp.maximum(ra[e, sl] + rb[e, sl], 0.0)
                return carry

            lax.fori_loop(0, _CH, ebody, 0)
            off = base + c * _CH
            pltpu.async_copy(hv, out_hbm.at[pl.ds(off, _CH)], so)

        issue(0, 0)
        issue(1, 1)
        n_pair = (n_ch - 1) // 2

        def pair(k, carry):
            c0 = 2 * k
            wait_gather(0)

            @pl.when(k > 0)
            def _():
                wait_store(0)

            compute_store(c0, 0)
            issue(c0 + 2, 0)
            wait_gather(1)

            @pl.when(k > 0)
            def _():
                wait_store(1)

            compute_store(c0 + 1, 1)

            @pl.when(k < n_pair - 1)
            def _():
                issue(c0 + 3, 1)

            return carry

        lax.fori_loop(0, n_pair, pair, 0)
        wait_gather(0)
        wait_store(0)
        compute_store(n_ch - 1, 0)
        wait_store(1)
        wait_store(0)

    return gather_k


def kernel(node_embeddings, edge_index, num_nodes, fiedler_vector,
           W0, b0, W1, b1, W2, b2):
    n, h = node_embeddings.shape
    n_edges = edge_index.shape[1]
    f32 = jnp.float32

    fcol = fiedler_vector.reshape(n, 1)
    w0a = W0[:h]
    w0b = W0[h:2 * h]
    ws = W0[2 * h].reshape(1, h)
    wd = W0[2 * h + 1].reshape(1, h)
    bf16 = jnp.bfloat16
    a_tab, b_tab = pl.pallas_call(
        _precompute_body,
        out_shape=(jax.ShapeDtypeStruct((n, h), f32),
                   jax.ShapeDtypeStruct((n, h), f32)),
    )(node_embeddings, fcol, w0a, w0b, ws, wd, b0.reshape(1, h))

    src = edge_index[0]
    dst = edge_index[1]
    h0 = _make_sc_gather(n_edges)(a_tab, b_tab, src, dst)

    w1p = W1.astype(bf16)
    n_blk = n_edges // _BE
    scores2d = pl.pallas_call(
        _mlp_body,
        grid=(n_blk,),
        in_specs=[
            pl.BlockSpec((_BE, h), lambda i: (i, 0)),
            pl.BlockSpec((h, h), lambda i: (0, 0)),
            pl.BlockSpec((1, h), lambda i: (0, 0)),
            pl.BlockSpec((h, 1), lambda i: (0, 0)),
            pl.BlockSpec((1, 1), lambda i: (0, 0)),
        ],
        out_specs=pl.BlockSpec((_BE, 1), lambda i: (i, 0)),
        out_shape=jax.ShapeDtypeStruct((n_edges, 1), f32),
    )(h0, w1p, b1.reshape(1, h), W2.astype(bf16), b2.reshape(1, 1))
    edge_scores = scores2d.reshape(n_edges)

    np_pad = ((n + _BI - 1) // _BI) * _BI
    fpad = jnp.concatenate(
        [fiedler_vector, jnp.full((np_pad - n,), jnp.inf, f32)])
    f2d = fpad.reshape(np_pad // _CJ, _CJ)
    num_pairs = min(_NUM_CAND, n * (n - 1) // 4)
    ck = jax.random.key(42)
    k1, k2 = jax.random.split(ck)
    idx1 = jax.random.randint(k1, (num_pairs,), 0, num_nodes // 2, jnp.int32)
    idx2 = jax.random.randint(k2, (num_pairs,), num_nodes // 2, num_nodes,
                              jnp.int32)
    pos = jnp.full((1, _PC), -1, jnp.int32)
    pos = pos.at[0, :num_pairs].set(idx1)
    pos = pos.at[0, _POS_OFF:_POS_OFF + num_pairs].set(idx2)

    sel = pl.pallas_call(
        functools.partial(_rank_body, n_j=np_pad // _CJ),
        grid=(np_pad // _BI,),
        in_specs=[
            pl.BlockSpec((_BI, 1), lambda i: (i, 0)),
            pl.BlockSpec((np_pad // _CJ, _CJ), lambda i: (0, 0)),
            pl.BlockSpec((1, _PC), lambda i: (0, 0)),
        ],
        out_specs=pl.BlockSpec((1, _PC), lambda i: (0, 0)),
        out_shape=jax.ShapeDtypeStruct((1, _PC), jnp.int32),
    )(fpad.reshape(np_pad, 1), f2d, pos)

    src_c = sel[0, :num_pairs]
    dst_c = sel[0, _POS_OFF:_POS_OFF + num_pairs]
    candidate_edges = jnp.stack([src_c, dst_c], axis=0)
    return edge_scores, candidate_edges


# speedup vs baseline: 5.3218x; 1.2054x over previous
import functools

import numpy as np

import jax
import jax.numpy as jnp
from jax import lax
from jax.experimental import pallas as pl
from jax.experimental.pallas import tpu as pltpu
from jax.experimental.pallas import tpu_sc as plsc

_H = 128
_NUM_CAND = 1000
_PC = 2048
_POS_OFF = 1024
_BI = 256
_CJ = 128
_BE = 8000
_NC, _NS = 2, 16
_NW = _NC * _NS
_CH = 80


def _precompute_body(ne, fcol, w0a, w0b, ws, wd, b0r, a_out, b_out):
    x = ne[...]
    f = fcol[...]
    a_out[...] = (jnp.dot(x, w0a[...], preferred_element_type=jnp.float32)
                  + f * ws[...] + b0r[...])
    b_out[...] = (jnp.dot(x, w0b[...], preferred_element_type=jnp.float32)
                  + f * wd[...])


def _mlp_body(h0, w1, b1r, w2, b2r, out):
    hb = h0[...].astype(jnp.bfloat16)
    h1 = jnp.maximum(
        jnp.dot(hb, w1[...], preferred_element_type=jnp.float32) + b1r[...], 0.0)
    out[...] = jnp.dot(h1.astype(jnp.bfloat16), w2[...],
                       preferred_element_type=jnp.float32) + b2r[...]


def _rank_body(fi_ref, f2d_ref, pos_ref, sel_ref, *, n_j):
    i = pl.program_id(0)
    fi = fi_ref[...]
    ii = i * _BI + lax.broadcasted_iota(jnp.int32, (_BI, 1), 0)
    d0 = i * (_BI // _CJ)

    def le_step(j, acc):
        return acc + (f2d_ref[pl.ds(j, 1), :] <= fi).astype(jnp.int32)

    def lt_step(j, acc):
        return acc + (f2d_ref[pl.ds(j, 1), :] < fi).astype(jnp.int32)

    acc = lax.fori_loop(0, d0, le_step, jnp.zeros((_BI, _CJ), jnp.int32))
    for t in range(_BI // _CJ):
        j = d0 + t
        fj = f2d_ref[pl.ds(j, 1), :]
        jidx = j * _CJ + lax.broadcasted_iota(jnp.int32, (1, _CJ), 1)
        tie = (fj == fi) & (jidx < ii)
        acc = acc + ((fj < fi) | tie).astype(jnp.int32)
    acc = lax.fori_loop(d0 + _BI // _CJ, n_j, lt_step, acc)
    rank = jnp.sum(acc, axis=1, keepdims=True)

    @pl.when(i == 0)
    def _():
        sel_ref[...] = jnp.zeros((1, _PC), jnp.int32)

    for c in range(_PC // 128):
        sl = slice(c * 128, (c + 1) * 128)
        match = rank == pos_ref[:, sl]
        vals = jnp.where(match, ii, 0)
        sel_ref[:, sl] = sel_ref[:, sl] + jnp.sum(vals, axis=0, keepdims=True)


def _make_sc_gather(n_edges):
    per_w = n_edges // _NW
    n_ch = per_w // _CH
    assert n_ch % 2 == 1 and n_ch >= 5
    mesh = plsc.VectorSubcoreMesh(core_axis_name="c", subcore_axis_name="s")

    @functools.partial(
        pl.kernel,
        mesh=mesh,
        out_type=jax.ShapeDtypeStruct((n_edges, _H), jnp.float32),
        scratch_types=[
            pltpu.VMEM((per_w,), jnp.int32), pltpu.VMEM((per_w,), jnp.int32),
            pltpu.VMEM((_CH, _H), jnp.float32), pltpu.VMEM((_CH, _H), jnp.float32),
            pltpu.VMEM((_CH, _H), jnp.float32), pltpu.VMEM((_CH, _H), jnp.float32),
            pltpu.VMEM((_CH, _H), jnp.float32), pltpu.VMEM((_CH, _H), jnp.float32),
            pltpu.SemaphoreType.DMA, pltpu.SemaphoreType.DMA,
            pltpu.SemaphoreType.DMA, pltpu.SemaphoreType.DMA,
            pltpu.SemaphoreType.DMA, pltpu.SemaphoreType.DMA,
        ],
    )
    def gather_k(a_hbm, b_hbm, src_hbm, dst_hbm, out_hbm,
                 si_all, di_all, ra0, rb0, ra1, rb1, hv0, hv1,
                 sa0, sb0, sa1, sb1, so0, so1):
        wid = lax.axis_index("s") * _NC + lax.axis_index("c")
        base = wid * per_w
        pltpu.sync_copy(src_hbm.at[pl.ds(base, per_w)], si_all)
        pltpu.sync_copy(dst_hbm.at[pl.ds(base, per_w)], di_all)

        bufs = ((ra0, rb0, hv0, sa0, sb0, so0),
                (ra1, rb1, hv1, sa1, sb1, so1))

        def issue(c, p):
            ra, rb, _, sa, sb, _ = bufs[p]
            isl = pl.ds(c * _CH, _CH)
            pltpu.async_copy(a_hbm.at[si_all.at[isl]], ra, sa)
            pltpu.async_copy(b_hbm.at[di_all.at[isl]], rb, sb)

        def wait_gather(p):
            ra, rb, _, sa, sb, _ = bufs[p]
            pltpu.make_async_copy(a_hbm.at[si_all.at[pl.ds(0, _CH)]], ra, sa).wait()
            pltpu.make_async_copy(b_hbm.at[di_all.at[pl.ds(0, _CH)]], rb, sb).wait()

        def wait_store(p):
            _, _, hv, _, _, so = bufs[p]
            pltpu.make_async_copy(
                hv, out_hbm.at[pl.ds(base, _CH)], so).wait()

        def compute_store(c, p):
            ra, rb, hv, _, _, so = bufs[p]

            def ebody(e, carry):
                for u in range(_H // 16):
                    sl = pl.ds(u * 16, 16)
                    hv[e, sl] = jnp.maximum(ra[e, sl] + rb[e, sl], 0.0)
                return carry

            lax.fori_loop(0, _CH, ebody, 0)
            off = base + c * _CH
            pltpu.async_copy(hv, out_hbm.at[pl.ds(off, _CH)], so)

        issue(0, 0)
        issue(1, 1)
        n_pair = (n_ch - 1) // 2

        def pair(k, carry):
            c0 = 2 * k
            wait_gather(0)

            @pl.when(k > 0)
            def _():
                wait_store(0)

            compute_store(c0, 0)
            issue(c0 + 2, 0)
            wait_gather(1)

            @pl.when(k > 0)
            def _():
                wait_store(1)

            compute_store(c0 + 1, 1)

            @pl.when(k < n_pair - 1)
            def _():
                issue(c0 + 3, 1)

            return carry

        lax.fori_loop(0, n_pair, pair, 0)
        wait_gather(0)
        wait_store(0)
        compute_store(n_ch - 1, 0)
        wait_store(1)
        wait_store(0)

    return gather_k


def kernel(node_embeddings, edge_index, num_nodes, fiedler_vector,
           W0, b0, W1, b1, W2, b2):
    n, h = node_embeddings.shape
    n_edges = edge_index.shape[1]
    f32 = jnp.float32

    fcol = fiedler_vector.reshape(n, 1)
    w0a = W0[:h]
    w0b = W0[h:2 * h]
    ws = W0[2 * h].reshape(1, h)
    wd = W0[2 * h + 1].reshape(1, h)
    bf16 = jnp.bfloat16
    a_tab, b_tab = pl.pallas_call(
        _precompute_body,
        out_shape=(jax.ShapeDtypeStruct((n, h), f32),
                   jax.ShapeDtypeStruct((n, h), f32)),
    )(node_embeddings, fcol, w0a, w0b, ws, wd, b0.reshape(1, h))

    src = edge_index[0]
    dst = edge_index[1]
    h0 = _make_sc_gather(n_edges)(a_tab, b_tab, src, dst)

    w1p = W1.astype(bf16)
    n_blk = n_edges // _BE
    scores2d = pl.pallas_call(
        _mlp_body,
        grid=(n_blk,),
        in_specs=[
            pl.BlockSpec((_BE, h), lambda i: (i, 0)),
            pl.BlockSpec((h, h), lambda i: (0, 0)),
            pl.BlockSpec((1, h), lambda i: (0, 0)),
            pl.BlockSpec((h, 1), lambda i: (0, 0)),
            pl.BlockSpec((1, 1), lambda i: (0, 0)),
        ],
        out_specs=pl.BlockSpec((_BE, 1), lambda i: (i, 0)),
        out_shape=jax.ShapeDtypeStruct((n_edges, 1), f32),
    )(h0, w1p, b1.reshape(1, h), W2.astype(bf16), b2.reshape(1, 1))
    edge_scores = scores2d.reshape(n_edges)

    np_pad = ((n + _BI - 1) // _BI) * _BI
    fpad = jnp.concatenate(
        [fiedler_vector, jnp.full((np_pad - n,), jnp.inf, f32)])
    f2d = fpad.reshape(np_pad // _CJ, _CJ)
    num_pairs = min(_NUM_CAND, n * (n - 1) // 4)
    ck = jax.random.key(42)
    k1, k2 = jax.random.split(ck)
    idx1 = jax.random.randint(k1, (num_pairs,), 0, num_nodes // 2, jnp.int32)
    idx2 = jax.random.randint(k2, (num_pairs,), num_nodes // 2, num_nodes,
                              jnp.int32)
    fill = jnp.full((_POS_OFF - num_pairs,), -1, jnp.int32)
    pos = jnp.concatenate([idx1, fill, idx2, fill]).reshape(1, _PC)

    sel = pl.pallas_call(
        functools.partial(_rank_body, n_j=np_pad // _CJ),
        grid=(np_pad // _BI,),
        in_specs=[
            pl.BlockSpec((_BI, 1), lambda i: (i, 0)),
            pl.BlockSpec((np_pad // _CJ, _CJ), lambda i: (0, 0)),
            pl.BlockSpec((1, _PC), lambda i: (0, 0)),
        ],
        out_specs=pl.BlockSpec((1, _PC), lambda i: (0, 0)),
        out_shape=jax.ShapeDtypeStruct((1, _PC), jnp.int32),
    )(fpad.reshape(np_pad, 1), f2d, pos)

    src_c = sel[0, :num_pairs]
    dst_c = sel[0, _POS_OFF:_POS_OFF + num_pairs]
    candidate_edges = jnp.stack([src_c, dst_c], axis=0)
    return edge_scores, candidate_edges


# speedup vs baseline: 6.4290x; 1.2080x over previous
import functools

import numpy as np

import jax
import jax.numpy as jnp
from jax import lax
from jax.experimental import pallas as pl
from jax.experimental.pallas import tpu as pltpu
from jax.experimental.pallas import tpu_sc as plsc

_H = 128
_NUM_CAND = 1000
_PC = 2048
_POS_OFF = 1024
_BI = 256
_CJ = 128
_BE = 16000
_NC, _NS = 2, 16
_NW = _NC * _NS
_CH = 80


def _precompute_body(ne, fcol, w0a, w0b, ws, wd, b0r, a_out, b_out):
    x = ne[...]
    f = fcol[...]
    a_out[...] = (jnp.dot(x, w0a[...], preferred_element_type=jnp.float32)
                  + f * ws[...] + b0r[...])
    b_out[...] = (jnp.dot(x, w0b[...], preferred_element_type=jnp.float32)
                  + f * wd[...])


def _mlp_body(h0, w1, b1r, w2, b2r, out):
    hb = h0[...].astype(jnp.bfloat16)
    h1 = jnp.maximum(
        jnp.dot(hb, w1[...], preferred_element_type=jnp.float32) + b1r[...], 0.0)
    out[...] = jnp.dot(h1.astype(jnp.bfloat16), w2[...],
                       preferred_element_type=jnp.float32) + b2r[...]


def _rank_body(fi_ref, f2d_ref, pos_ref, sel_ref, *, n_j):
    i = pl.program_id(0)
    fi = fi_ref[...]
    ii = i * _BI + lax.broadcasted_iota(jnp.int32, (_BI, 1), 0)
    d0 = i * (_BI // _CJ)

    def le_step(j, acc):
        return acc + (f2d_ref[pl.ds(j, 1), :] <= fi).astype(jnp.int32)

    def le_step4(j4, acc):
        for t in range(4):
            acc = acc + (f2d_ref[pl.ds(j4 * 4 + t, 1), :] <= fi).astype(jnp.int32)
        return acc

    def lt_step(j, acc):
        return acc + (f2d_ref[pl.ds(j, 1), :] < fi).astype(jnp.int32)

    def lt_step4(j4, acc):
        for t in range(4):
            acc = acc + (f2d_ref[pl.ds(j4 * 4 + t, 1), :] < fi).astype(jnp.int32)
        return acc

    acc = jnp.zeros((_BI, _CJ), jnp.int32)
    acc = lax.fori_loop(0, d0 // 4, le_step4, acc)
    acc = lax.fori_loop(d0 // 4 * 4, d0, le_step, acc)
    for t in range(_BI // _CJ):
        j = d0 + t
        fj = f2d_ref[pl.ds(j, 1), :]
        jidx = j * _CJ + lax.broadcasted_iota(jnp.int32, (1, _CJ), 1)
        tie = (fj == fi) & (jidx < ii)
        acc = acc + ((fj < fi) | tie).astype(jnp.int32)
    d1 = d0 + _BI // _CJ
    d1c = (d1 + 3) // 4
    acc = lax.fori_loop(d1, jnp.minimum(d1c * 4, n_j), lt_step, acc)
    acc = lax.fori_loop(d1c, n_j // 4, lt_step4, acc)
    acc = lax.fori_loop(n_j // 4 * 4, n_j, lt_step, acc)
    rank = jnp.sum(acc, axis=1, keepdims=True)

    @pl.when(i == 0)
    def _():
        sel_ref[...] = jnp.zeros((1, _PC), jnp.int32)

    for c in range(_PC // 128):
        sl = slice(c * 128, (c + 1) * 128)
        match = rank == pos_ref[:, sl]
        vals = jnp.where(match, ii, 0)
        sel_ref[:, sl] = sel_ref[:, sl] + jnp.sum(vals, axis=0, keepdims=True)


def _make_sc_gather(n_edges):
    per_w = n_edges // _NW
    n_ch = per_w // _CH
    assert n_ch % 2 == 1 and n_ch >= 5
    mesh = plsc.VectorSubcoreMesh(core_axis_name="c", subcore_axis_name="s")

    @functools.partial(
        pl.kernel,
        mesh=mesh,
        out_type=jax.ShapeDtypeStruct((n_edges, _H), jnp.float32),
        scratch_types=[
            pltpu.VMEM((per_w,), jnp.int32), pltpu.VMEM((per_w,), jnp.int32),
            pltpu.VMEM((_CH, _H), jnp.float32), pltpu.VMEM((_CH, _H), jnp.float32),
            pltpu.VMEM((_CH, _H), jnp.float32), pltpu.VMEM((_CH, _H), jnp.float32),
            pltpu.VMEM((_CH, _H), jnp.float32), pltpu.VMEM((_CH, _H), jnp.float32),
            pltpu.SemaphoreType.DMA, pltpu.SemaphoreType.DMA,
            pltpu.SemaphoreType.DMA, pltpu.SemaphoreType.DMA,
            pltpu.SemaphoreType.DMA, pltpu.SemaphoreType.DMA,
        ],
    )
    def gather_k(a_hbm, b_hbm, src_hbm, dst_hbm, out_hbm,
                 si_all, di_all, ra0, rb0, ra1, rb1, hv0, hv1,
                 sa0, sb0, sa1, sb1, so0, so1):
        wid = lax.axis_index("s") * _NC + lax.axis_index("c")
        base = wid * per_w
        pltpu.sync_copy(src_hbm.at[pl.ds(base, per_w)], si_all)
        pltpu.sync_copy(dst_hbm.at[pl.ds(base, per_w)], di_all)

        bufs = ((ra0, rb0, hv0, sa0, sb0, so0),
                (ra1, rb1, hv1, sa1, sb1, so1))

        def issue(c, p):
            ra, rb, _, sa, sb, _ = bufs[p]
            isl = pl.ds(c * _CH, _CH)
            pltpu.async_copy(a_hbm.at[si_all.at[isl]], ra, sa)
            pltpu.async_copy(b_hbm.at[di_all.at[isl]], rb, sb)

        def wait_gather(p):
            ra, rb, _, sa, sb, _ = bufs[p]
            pltpu.make_async_copy(a_hbm.at[si_all.at[pl.ds(0, _CH)]], ra, sa).wait()
            pltpu.make_async_copy(b_hbm.at[di_all.at[pl.ds(0, _CH)]], rb, sb).wait()

        def wait_store(p):
            _, _, hv, _, _, so = bufs[p]
            pltpu.make_async_copy(
                hv, out_hbm.at[pl.ds(base, _CH)], so).wait()

        def compute_store(c, p):
            ra, rb, hv, _, _, so = bufs[p]

            def ebody(e, carry):
                for u in range(_H // 16):
                    sl = pl.ds(u * 16, 16)
                    hv[e, sl] = jnp.maximum(ra[e, sl] + rb[e, sl], 0.0)
                return carry

            lax.fori_loop(0, _CH, ebody, 0)
            off = base + c * _CH
            pltpu.async_copy(hv, out_hbm.at[pl.ds(off, _CH)], so)

        issue(0, 0)
        issue(1, 1)
        n_pair = (n_ch - 1) // 2

        def pair(k, carry):
            c0 = 2 * k
            wait_gather(0)

            @pl.when(k > 0)
            def _():
                wait_store(0)

            compute_store(c0, 0)
            issue(c0 + 2, 0)
            wait_gather(1)

            @pl.when(k > 0)
            def _():
                wait_store(1)

            compute_store(c0 + 1, 1)

            @pl.when(k < n_pair - 1)
            def _():
                issue(c0 + 3, 1)

            return carry

        lax.fori_loop(0, n_pair, pair, 0)
        wait_gather(0)
        wait_store(0)
        compute_store(n_ch - 1, 0)
        wait_store(1)
        wait_store(0)

    return gather_k


def kernel(node_embeddings, edge_index, num_nodes, fiedler_vector,
           W0, b0, W1, b1, W2, b2):
    n, h = node_embeddings.shape
    n_edges = edge_index.shape[1]
    f32 = jnp.float32

    fcol = fiedler_vector.reshape(n, 1)
    w0a = W0[:h]
    w0b = W0[h:2 * h]
    ws = W0[2 * h].reshape(1, h)
    wd = W0[2 * h + 1].reshape(1, h)
    bf16 = jnp.bfloat16
    a_tab, b_tab = pl.pallas_call(
        _precompute_body,
        out_shape=(jax.ShapeDtypeStruct((n, h), f32),
                   jax.ShapeDtypeStruct((n, h), f32)),
    )(node_embeddings, fcol, w0a, w0b, ws, wd, b0.reshape(1, h))

    src = edge_index[0]
    dst = edge_index[1]
    h0 = _make_sc_gather(n_edges)(a_tab, b_tab, src, dst)

    w1p = W1.astype(bf16)
    n_blk = n_edges // _BE
    scores2d = pl.pallas_call(
        _mlp_body,
        grid=(n_blk,),
        in_specs=[
            pl.BlockSpec((_BE, h), lambda i: (i, 0)),
            pl.BlockSpec((h, h), lambda i: (0, 0)),
            pl.BlockSpec((1, h), lambda i: (0, 0)),
            pl.BlockSpec((h, 1), lambda i: (0, 0)),
            pl.BlockSpec((1, 1), lambda i: (0, 0)),
        ],
        out_specs=pl.BlockSpec((_BE, 1), lambda i: (i, 0)),
        out_shape=jax.ShapeDtypeStruct((n_edges, 1), f32),
    )(h0, w1p, b1.reshape(1, h), W2.astype(bf16), b2.reshape(1, 1))
    edge_scores = scores2d.reshape(n_edges)

    np_pad = ((n + _BI - 1) // _BI) * _BI
    fpad = jnp.concatenate(
        [fiedler_vector, jnp.full((np_pad - n,), jnp.inf, f32)])
    f2d = fpad.reshape(np_pad // _CJ, _CJ)
    num_pairs = min(_NUM_CAND, n * (n - 1) // 4)
    ck = jax.random.key(42)
    k1, k2 = jax.random.split(ck)
    idx1 = jax.random.randint(k1, (num_pairs,), 0, num_nodes // 2, jnp.int32)
    idx2 = jax.random.randint(k2, (num_pairs,), num_nodes // 2, num_nodes,
                              jnp.int32)
    fill = jnp.full((_POS_OFF - num_pairs,), -1, jnp.int32)
    pos = jnp.concatenate([idx1, fill, idx2, fill]).reshape(1, _PC)

    sel = pl.pallas_call(
        functools.partial(_rank_body, n_j=np_pad // _CJ),
        grid=(np_pad // _BI,),
        in_specs=[
            pl.BlockSpec((_BI, 1), lambda i: (i, 0)),
            pl.BlockSpec((np_pad // _CJ, _CJ), lambda i: (0, 0)),
            pl.BlockSpec((1, _PC), lambda i: (0, 0)),
        ],
        out_specs=pl.BlockSpec((1, _PC), lambda i: (0, 0)),
        out_shape=jax.ShapeDtypeStruct((1, _PC), jnp.int32),
    )(fpad.reshape(np_pad, 1), f2d, pos)

    src_c = sel[0, :num_pairs]
    dst_c = sel[0, _POS_OFF:_POS_OFF + num_pairs]
    candidate_edges = jnp.stack([src_c, dst_c], axis=0)
    return edge_scores, candidate_edges


# speedup vs baseline: 6.4303x; 1.0002x over previous
import functools

import numpy as np

import jax
import jax.numpy as jnp
from jax import lax
from jax.experimental import pallas as pl
from jax.experimental.pallas import tpu as pltpu
from jax.experimental.pallas import tpu_sc as plsc

_H = 128
_NUM_CAND = 1000
_PC = 2048
_POS_OFF = 1024
_BI = 256
_CJ = 128
_BE = 20000
_NC, _NS = 2, 16
_NW = _NC * _NS
_CH = 80


def _precompute_body(ne, fcol, w0a, w0b, ws, wd, b0r, a_out, b_out):
    x = ne[...]
    f = fcol[...]
    a_out[...] = (jnp.dot(x, w0a[...], preferred_element_type=jnp.float32)
                  + f * ws[...] + b0r[...])
    b_out[...] = (jnp.dot(x, w0b[...], preferred_element_type=jnp.float32)
                  + f * wd[...])


def _mlp_body(h0, w1, b1r, w2, b2r, out):
    hb = h0[...].astype(jnp.bfloat16)
    h1 = jnp.maximum(
        jnp.dot(hb, w1[...], preferred_element_type=jnp.float32) + b1r[...], 0.0)
    out[...] = jnp.dot(h1.astype(jnp.bfloat16), w2[...],
                       preferred_element_type=jnp.float32) + b2r[...]


def _rank_body(fi_ref, f2d_ref, pos_ref, sel_ref, *, n_j):
    i = pl.program_id(0)
    fi = fi_ref[...]
    ii = i * _BI + lax.broadcasted_iota(jnp.int32, (_BI, 1), 0)
    d0 = i * (_BI // _CJ)

    def le_step(j, acc):
        return acc + (f2d_ref[pl.ds(j, 1), :] <= fi).astype(jnp.int32)

    def le_step4(j4, acc):
        for t in range(4):
            acc = acc + (f2d_ref[pl.ds(j4 * 4 + t, 1), :] <= fi).astype(jnp.int32)
        return acc

    def lt_step(j, acc):
        return acc + (f2d_ref[pl.ds(j, 1), :] < fi).astype(jnp.int32)

    def lt_step4(j4, acc):
        for t in range(4):
            acc = acc + (f2d_ref[pl.ds(j4 * 4 + t, 1), :] < fi).astype(jnp.int32)
        return acc

    acc = jnp.zeros((_BI, _CJ), jnp.int32)
    acc = lax.fori_loop(0, d0 // 4, le_step4, acc)
    acc = lax.fori_loop(d0 // 4 * 4, d0, le_step, acc)
    for t in range(_BI // _CJ):
        j = d0 + t
        fj = f2d_ref[pl.ds(j, 1), :]
        jidx = j * _CJ + lax.broadcasted_iota(jnp.int32, (1, _CJ), 1)
        tie = (fj == fi) & (jidx < ii)
        acc = acc + ((fj < fi) | tie).astype(jnp.int32)
    d1 = d0 + _BI // _CJ
    d1c = (d1 + 3) // 4
    acc = lax.fori_loop(d1, jnp.minimum(d1c * 4, n_j), lt_step, acc)
    acc = lax.fori_loop(d1c, n_j // 4, lt_step4, acc)
    acc = lax.fori_loop(n_j // 4 * 4, n_j, lt_step, acc)
    rank = jnp.sum(acc, axis=1, keepdims=True)

    @pl.when(i == 0)
    def _():
        sel_ref[...] = jnp.zeros((1, _PC), jnp.int32)

    for c in range(_PC // 128):
        sl = slice(c * 128, (c + 1) * 128)
        match = rank == pos_ref[:, sl]
        vals = jnp.where(match, ii, 0)
        sel_ref[:, sl] = sel_ref[:, sl] + jnp.sum(vals, axis=0, keepdims=True)


def _make_sc_gather(n_edges):
    per_w = n_edges // _NW
    n_ch = per_w // _CH
    assert n_ch % 2 == 1 and n_ch >= 5
    mesh = plsc.VectorSubcoreMesh(core_axis_name="c", subcore_axis_name="s")

    @functools.partial(
        pl.kernel,
        mesh=mesh,
        out_type=jax.ShapeDtypeStruct((n_edges, _H), jnp.float32),
        scratch_types=[
            pltpu.VMEM((per_w,), jnp.int32), pltpu.VMEM((per_w,), jnp.int32),
            pltpu.VMEM((_CH, _H), jnp.float32), pltpu.VMEM((_CH, _H), jnp.float32),
            pltpu.VMEM((_CH, _H), jnp.float32), pltpu.VMEM((_CH, _H), jnp.float32),
            pltpu.VMEM((_CH, _H), jnp.float32), pltpu.VMEM((_CH, _H), jnp.float32),
            pltpu.SemaphoreType.DMA, pltpu.SemaphoreType.DMA,
            pltpu.SemaphoreType.DMA, pltpu.SemaphoreType.DMA,
            pltpu.SemaphoreType.DMA, pltpu.SemaphoreType.DMA,
        ],
    )
    def gather_k(a_hbm, b_hbm, src_hbm, dst_hbm, out_hbm,
                 si_all, di_all, ra0, rb0, ra1, rb1, hv0, hv1,
                 sa0, sb0, sa1, sb1, so0, so1):
        wid = lax.axis_index("s") * _NC + lax.axis_index("c")
        base = wid * per_w
        pltpu.sync_copy(src_hbm.at[pl.ds(base, per_w)], si_all)
        pltpu.sync_copy(dst_hbm.at[pl.ds(base, per_w)], di_all)

        bufs = ((ra0, rb0, hv0, sa0, sb0, so0),
                (ra1, rb1, hv1, sa1, sb1, so1))

        def issue(c, p):
            ra, rb, _, sa, sb, _ = bufs[p]
            isl = pl.ds(c * _CH, _CH)
            pltpu.async_copy(a_hbm.at[si_all.at[isl]], ra, sa)
            pltpu.async_copy(b_hbm.at[di_all.at[isl]], rb, sb)

        def wait_gather(p):
            ra, rb, _, sa, sb, _ = bufs[p]
            pltpu.make_async_copy(a_hbm.at[si_all.at[pl.ds(0, _CH)]], ra, sa).wait()
            pltpu.make_async_copy(b_hbm.at[di_all.at[pl.ds(0, _CH)]], rb, sb).wait()

        def wait_store(p):
            _, _, hv, _, _, so = bufs[p]
            pltpu.make_async_copy(
                hv, out_hbm.at[pl.ds(base, _CH)], so).wait()

        def compute_store(c, p):
            ra, rb, hv, _, _, so = bufs[p]

            def ebody(e, carry):
                for u in range(_H // 16):
                    sl = pl.ds(u * 16, 16)
                    hv[e, sl] = jnp.maximum(ra[e, sl] + rb[e, sl], 0.0)
                return carry

            lax.fori_loop(0, _CH, ebody, 0)
            off = base + c * _CH
            pltpu.async_copy(hv, out_hbm.at[pl.ds(off, _CH)], so)

        issue(0, 0)
        issue(1, 1)
        n_pair = (n_ch - 1) // 2

        def pair(k, carry):
            c0 = 2 * k
            wait_gather(0)

            @pl.when(k > 0)
            def _():
                wait_store(0)

            compute_store(c0, 0)
            issue(c0 + 2, 0)
            wait_gather(1)

            @pl.when(k > 0)
            def _():
                wait_store(1)

            compute_store(c0 + 1, 1)

            @pl.when(k < n_pair - 1)
            def _():
                issue(c0 + 3, 1)

            return carry

        lax.fori_loop(0, n_pair, pair, 0)
        wait_gather(0)
        wait_store(0)
        compute_store(n_ch - 1, 0)
        wait_store(1)
        wait_store(0)

    return gather_k


def kernel(node_embeddings, edge_index, num_nodes, fiedler_vector,
           W0, b0, W1, b1, W2, b2):
    n, h = node_embeddings.shape
    n_edges = edge_index.shape[1]
    f32 = jnp.float32

    fcol = fiedler_vector.reshape(n, 1)
    w0a = W0[:h]
    w0b = W0[h:2 * h]
    ws = W0[2 * h].reshape(1, h)
    wd = W0[2 * h + 1].reshape(1, h)
    bf16 = jnp.bfloat16
    a_tab, b_tab = pl.pallas_call(
        _precompute_body,
        out_shape=(jax.ShapeDtypeStruct((n, h), f32),
                   jax.ShapeDtypeStruct((n, h), f32)),
    )(node_embeddings, fcol, w0a, w0b, ws, wd, b0.reshape(1, h))

    src = edge_index[0]
    dst = edge_index[1]
    h0 = _make_sc_gather(n_edges)(a_tab, b_tab, src, dst)

    w1p = W1.astype(bf16)
    n_blk = n_edges // _BE
    scores2d = pl.pallas_call(
        _mlp_body,
        grid=(n_blk,),
        in_specs=[
            pl.BlockSpec((_BE, h), lambda i: (i, 0)),
            pl.BlockSpec((h, h), lambda i: (0, 0)),
            pl.BlockSpec((1, h), lambda i: (0, 0)),
            pl.BlockSpec((h, 1), lambda i: (0, 0)),
            pl.BlockSpec((1, 1), lambda i: (0, 0)),
        ],
        out_specs=pl.BlockSpec((_BE, 1), lambda i: (i, 0)),
        out_shape=jax.ShapeDtypeStruct((n_edges, 1), f32),
    )(h0, w1p, b1.reshape(1, h), W2.astype(bf16), b2.reshape(1, 1))
    edge_scores = scores2d.reshape(n_edges)

    np_pad = ((n + _BI - 1) // _BI) * _BI
    fpad = jnp.concatenate(
        [fiedler_vector, jnp.full((np_pad - n,), jnp.inf, f32)])
    f2d = fpad.reshape(np_pad // _CJ, _CJ)
    num_pairs = min(_NUM_CAND, n * (n - 1) // 4)
    ck = jax.random.key(42)
    k1, k2 = jax.random.split(ck)
    idx1 = jax.random.randint(k1, (num_pairs,), 0, num_nodes // 2, jnp.int32)
    idx2 = jax.random.randint(k2, (num_pairs,), num_nodes // 2, num_nodes,
                              jnp.int32)
    fill = jnp.full((_POS_OFF - num_pairs,), -1, jnp.int32)
    pos = jnp.concatenate([idx1, fill, idx2, fill]).reshape(1, _PC)

    sel = pl.pallas_call(
        functools.partial(_rank_body, n_j=np_pad // _CJ),
        grid=(np_pad // _BI,),
        in_specs=[
            pl.BlockSpec((_BI, 1), lambda i: (i, 0)),
            pl.BlockSpec((np_pad // _CJ, _CJ), lambda i: (0, 0)),
            pl.BlockSpec((1, _PC), lambda i: (0, 0)),
        ],
        out_specs=pl.BlockSpec((1, _PC), lambda i: (0, 0)),
        out_shape=jax.ShapeDtypeStruct((1, _PC), jnp.int32),
    )(fpad.reshape(np_pad, 1), f2d, pos)

    src_c = sel[0, :num_pairs]
    dst_c = sel[0, _POS_OFF:_POS_OFF + num_pairs]
    candidate_edges = jnp.stack([src_c, dst_c], axis=0)
    return edge_scores, candidate_edges
